# baseline (speedup 1.0000x reference)
import functools

import jax
import jax.numpy as jnp
from jax import lax
from jax.experimental import pallas as pl
from jax.experimental.pallas import tpu as pltpu
from jax.experimental.pallas import tpu_sc as plsc

NUM_FIELDS = 26
VOCAB = 100000
EMBED_DIM = 32
BATCH = 16384

N_ROWS = BATCH * NUM_FIELDS
NUM_WORKERS = 32
PER_W = N_ROWS // NUM_WORKERS
R = 832
NCHUNK = PER_W // R
LANES = 16


def _body(cat_hbm, tab_hbm, col_hbm, out_hbm, idx_v, rows_v, bias_v, col_v,
          offs_v, sem):
    nc = 2
    wid = lax.axis_index("s") * nc + lax.axis_index("c")
    base = wid * PER_W

    pltpu.sync_copy(col_hbm, col_v)

    lanes = lax.iota(jnp.int32, LANES)
    for i in range(R // LANES):
        offs_v[pl.ds(i * LANES, LANES)] = ((lanes + i * LANES) % NUM_FIELDS) * VOCAB

    def bias_body(rr, carry):
        f = rr % NUM_FIELDS
        bias_v[rr, pl.ds(0, LANES)] = col_v[f, pl.ds(0, LANES)]
        bias_v[rr, pl.ds(LANES, LANES)] = col_v[f, pl.ds(LANES, LANES)]
        return carry

    lax.fori_loop(0, R, bias_body, 0)

    def chunk_body(c, carry):
        cbase = base + c * R
        pltpu.sync_copy(cat_hbm.at[pl.ds(cbase, R)], idx_v)

        def flat_body(i, c2):
            s = pl.ds(i * LANES, LANES)
            idx_v[s] = idx_v[s] + offs_v[s]
            return c2

        lax.fori_loop(0, R // LANES, flat_body, 0)

        pltpu.async_copy(tab_hbm.at[idx_v], rows_v, sem).wait()

        def add_body(rr, c2):
            s0 = pl.ds(0, LANES)
            s1 = pl.ds(LANES, LANES)
            rows_v[rr, s0] = rows_v[rr, s0] + bias_v[rr, s0]
            rows_v[rr, s1] = rows_v[rr, s1] + bias_v[rr, s1]
            return c2

        lax.fori_loop(0, R, add_body, 0)

        pltpu.sync_copy(rows_v, out_hbm.at[pl.ds(cbase, R)])
        return carry

    lax.fori_loop(0, NCHUNK, chunk_body, 0)


@jax.jit
def _run(cat_flat, tab_flat, col):
    mesh = plsc.VectorSubcoreMesh(core_axis_name="c", subcore_axis_name="s")
    k = functools.partial(
        pl.kernel,
        mesh=mesh,
        out_type=jax.ShapeDtypeStruct((N_ROWS, EMBED_DIM), jnp.float32),
        scratch_types=[
            pltpu.VMEM((R,), jnp.int32),
            pltpu.VMEM((R, EMBED_DIM), jnp.float32),
            pltpu.VMEM((R, EMBED_DIM), jnp.float32),
            pltpu.VMEM((NUM_FIELDS, EMBED_DIM), jnp.float32),
            pltpu.VMEM((R,), jnp.int32),
            pltpu.SemaphoreType.DMA,
        ],
        compiler_params=pltpu.CompilerParams(use_tc_tiling_on_sc=False),
    )(_body)
    return k(cat_flat, tab_flat, col)


def kernel(categorical_inputs, tables, column_embedding):
    cat_flat = categorical_inputs.astype(jnp.int32).reshape(N_ROWS)
    tab_flat = tables.reshape(NUM_FIELDS * VOCAB, EMBED_DIM)
    out = _run(cat_flat, tab_flat, column_embedding)
    return out.reshape(BATCH, NUM_FIELDS, EMBED_DIM)


# speedup vs baseline: 4.1113x; 4.1113x over previous
import functools

import jax
import jax.numpy as jnp
from jax import lax
from jax.experimental import pallas as pl
from jax.experimental.pallas import tpu as pltpu
from jax.experimental.pallas import tpu_sc as plsc

NUM_FIELDS = 26
VOCAB = 100000
EMBED_DIM = 32
BATCH = 16384

NUM_LINES = NUM_FIELDS * EMBED_DIM
NUM_WORKERS = 32
LINES_PER_W = NUM_LINES // NUM_WORKERS
HALF = BATCH // 2
LANES = 16


def _body(cat_hbm, tab_hbm, col_hbm, out_hbm, line_v, idx_v, out_v, col_v, sem):
    nc = 2
    wid = lax.axis_index("s") * nc + lax.axis_index("c")
    base = wid * LINES_PER_W

    pltpu.sync_copy(col_hbm, col_v)

    def line_body(li, carry):
        l = base + li
        f = l // EMBED_DIM
        d = l % EMBED_DIM
        fidx = jnp.full((LANES,), 0, jnp.int32) + f
        didx = jnp.full((LANES,), 0, jnp.int32) + d
        bias = plsc.load_gather(col_v, [fidx, didx])
        pltpu.sync_copy(tab_hbm.at[l], line_v)

        def half_body(h, c2):
            hbase = h * HALF
            pltpu.sync_copy(cat_hbm.at[f, pl.ds(hbase, HALF)], idx_v)

            def gather_body(i, c3):
                s = pl.ds(i * LANES, LANES)
                idx16 = idx_v[s]
                vals = plsc.load_gather(line_v, [idx16])
                out_v[s] = vals + bias
                return c3

            lax.fori_loop(0, HALF // LANES, gather_body, 0)
            pltpu.sync_copy(out_v, out_hbm.at[l, pl.ds(hbase, HALF)])
            return c2

        lax.fori_loop(0, 2, half_body, 0)
        return carry

    lax.fori_loop(0, LINES_PER_W, line_body, 0)


@jax.jit
def _run(cat_t, tab_t, col):
    mesh = plsc.VectorSubcoreMesh(core_axis_name="c", subcore_axis_name="s")
    k = functools.partial(
        pl.kernel,
        mesh=mesh,
        out_type=jax.ShapeDtypeStruct((NUM_LINES, BATCH), jnp.float32),
        scratch_types=[
            pltpu.VMEM((VOCAB,), jnp.float32),
            pltpu.VMEM((HALF,), jnp.int32),
            pltpu.VMEM((HALF,), jnp.float32),
            pltpu.VMEM((NUM_FIELDS, EMBED_DIM), jnp.float32),
            pltpu.SemaphoreType.DMA,
        ],
        compiler_params=pltpu.CompilerParams(
            use_tc_tiling_on_sc=True, needs_layout_passes=False
        ),
    )(_body)
    return k(cat_t, tab_t, col)


def kernel(categorical_inputs, tables, column_embedding):
    cat_t = categorical_inputs.astype(jnp.int32).T
    tab_t = tables.transpose(0, 2, 1).reshape(NUM_LINES, VOCAB)
    out_t = _run(cat_t, tab_t, column_embedding)
    return out_t.reshape(NUM_FIELDS, EMBED_DIM, BATCH).transpose(2, 0, 1)
